# Initial kernel scaffold; baseline (speedup 1.0000x reference)
import jax
import jax.numpy as jnp
from jax.experimental import pallas as pl


def kernel(indices, embed_weight):
    raise NotImplementedError("write your pallas kernel here")


# speedup vs baseline: 2.7302x; 2.7302x over previous
import functools

import jax
import jax.numpy as jnp
from jax import lax
from jax.experimental import pallas as pl
from jax.experimental.pallas import tpu as pltpu
from jax.experimental.pallas import tpu_sc as plsc

VOCAB_DIM = 128
TOTAL = 4096 * 50
NUM_WORKERS = 32
CHUNK = 128
CHUNKS_PER_W = TOTAL // (NUM_WORKERS * CHUNK)

_mesh = plsc.VectorSubcoreMesh(core_axis_name="c", subcore_axis_name="s")


@functools.partial(
    pl.kernel,
    mesh=_mesh,
    out_type=jax.ShapeDtypeStruct(
        (NUM_WORKERS, CHUNKS_PER_W, CHUNK, VOCAB_DIM), jnp.float32
    ),
    scratch_types=[
        pltpu.VMEM((CHUNKS_PER_W, CHUNK), jnp.int32),
        pltpu.VMEM((CHUNK, VOCAB_DIM), jnp.float32),
        pltpu.SemaphoreType.DMA,
    ],
)
def _sc_gather(idx_hbm, table_hbm, out_hbm, idx_v, rows_v, sem):
    c = lax.axis_index("c")
    s = lax.axis_index("s")
    wid = s * 2 + c
    pltpu.sync_copy(idx_hbm.at[wid], idx_v)

    def step(j, carry):
        pltpu.async_copy(table_hbm.at[idx_v.at[j]], rows_v, sem).wait()
        pltpu.sync_copy(rows_v, out_hbm.at[wid, j])
        return carry

    lax.fori_loop(0, CHUNKS_PER_W, step, 0)


def kernel(indices, embed_weight):
    idx = indices.reshape(NUM_WORKERS, CHUNKS_PER_W, CHUNK).astype(jnp.int32)
    out = _sc_gather(idx, embed_weight)
    return out.reshape(indices.shape[0], indices.shape[1], VOCAB_DIM)


# speedup vs baseline: 2.7990x; 1.0252x over previous
import functools

import jax
import jax.numpy as jnp
from jax import lax
from jax.experimental import pallas as pl
from jax.experimental.pallas import tpu as pltpu
from jax.experimental.pallas import tpu_sc as plsc

EMBED_DIM = 128
TOTAL = 4096 * 50
NUM_WORKERS = 32
CHUNK = 128
STEPS = TOTAL // (NUM_WORKERS * CHUNK)
NBUF = 5
LOOKAHEAD = 3

_mesh = plsc.VectorSubcoreMesh(core_axis_name="c", subcore_axis_name="s")


@functools.partial(
    pl.kernel,
    mesh=_mesh,
    out_type=jax.ShapeDtypeStruct(
        (NUM_WORKERS, STEPS, CHUNK, EMBED_DIM), jnp.float32
    ),
    scratch_types=[
        pltpu.VMEM((STEPS, CHUNK), jnp.int32),
        pltpu.VMEM((NBUF, CHUNK, EMBED_DIM), jnp.float32),
        pltpu.SemaphoreType.DMA((NBUF,)),
        pltpu.SemaphoreType.DMA((NBUF,)),
    ],
)
def _sc_gather(idx_hbm, table_hbm, out_hbm, idx_v, bufs, gsem, wsem):
    c = lax.axis_index("c")
    s = lax.axis_index("s")
    wid = s * 2 + c
    pltpu.sync_copy(idx_hbm.at[wid], idx_v)

    def gstart(j, b):
        pltpu.async_copy(table_hbm.at[idx_v.at[j]], bufs.at[b], gsem.at[b])

    def gwait(j, b):
        pltpu.make_async_copy(
            table_hbm.at[idx_v.at[j]], bufs.at[b], gsem.at[b]
        ).wait()

    def wstart(j, b):
        pltpu.async_copy(bufs.at[b], out_hbm.at[wid, j], wsem.at[b])

    def wwait(j, b):
        pltpu.make_async_copy(
            bufs.at[b], out_hbm.at[wid, j], wsem.at[b]
        ).wait()

    for t in range(LOOKAHEAD):
        gstart(t, t)

    def body(jj, carry):
        j0 = jj * NBUF
        for t in range(NBUF):
            j = j0 + t
            bn = (t + LOOKAHEAD) % NBUF
            nj = j + LOOKAHEAD

            @pl.when(jnp.logical_and(nj < STEPS, j >= NBUF - LOOKAHEAD))
            def _():
                wwait(nj - NBUF, bn)

            @pl.when(nj < STEPS)
            def _():
                gstart(nj, bn)

            gwait(j, t)
            wstart(j, t)
        return carry

    lax.fori_loop(0, STEPS // NBUF, body, 0)

    for t in range(NBUF):
        wwait(STEPS - NBUF + t, t)


def kernel(indices, embed_weight):
    idx = indices.reshape(NUM_WORKERS, STEPS, CHUNK).astype(jnp.int32)
    out = _sc_gather(idx, embed_weight)
    return out.reshape(indices.shape[0], indices.shape[1], EMBED_DIM)


# speedup vs baseline: 6.5853x; 2.3527x over previous
import functools

import jax
import jax.numpy as jnp
from jax import lax
from jax.experimental import pallas as pl
from jax.experimental.pallas import tpu as pltpu
from jax.experimental.pallas import tpu_sc as plsc

EMBED_DIM = 128
HIST = 50
BATCH = 4096
NUM_WORKERS = 32
BPW = BATCH // NUM_WORKERS
STEPS = HIST
NBUF = 5
LOOK = 3

_mesh = plsc.VectorSubcoreMesh(core_axis_name="c", subcore_axis_name="s")


@functools.partial(
    pl.kernel,
    mesh=_mesh,
    out_type=jax.ShapeDtypeStruct((HIST, BATCH, EMBED_DIM), jnp.float32),
    scratch_types=[
        pltpu.VMEM((HIST, BPW), jnp.int32),
        pltpu.VMEM((NBUF, BPW, EMBED_DIM), jnp.float32),
        pltpu.SemaphoreType.DMA((NBUF,)),
        pltpu.SemaphoreType.DMA((NBUF,)),
    ],
)
def _sc_gather(idx_hbm, table_hbm, out_hbm, idx_v, bufs, gsem, wsem):
    c = lax.axis_index("c")
    s = lax.axis_index("s")
    wid = s * 2 + c
    b0 = wid * BPW
    pltpu.sync_copy(idx_hbm.at[:, pl.ds(b0, BPW)], idx_v)

    def gstart(j, b):
        pltpu.async_copy(table_hbm.at[idx_v.at[j]], bufs.at[b], gsem.at[b])

    def gwait(j, b):
        pltpu.make_async_copy(
            table_hbm.at[idx_v.at[j]], bufs.at[b], gsem.at[b]
        ).wait()

    def wstart(j, b):
        pltpu.async_copy(
            bufs.at[b], out_hbm.at[j, pl.ds(b0, BPW)], wsem.at[b]
        )

    def wwait(j, b):
        pltpu.make_async_copy(
            bufs.at[b], out_hbm.at[j, pl.ds(b0, BPW)], wsem.at[b]
        ).wait()

    for t in range(LOOK):
        gstart(t, t)

    def body(jj, carry):
        j0 = jj * NBUF
        for t in range(NBUF):
            j = j0 + t
            bn = (t + LOOK) % NBUF
            nj = j + LOOK

            @pl.when(jnp.logical_and(nj < STEPS, j >= NBUF - LOOK))
            def _():
                wwait(nj - NBUF, bn)

            @pl.when(nj < STEPS)
            def _():
                gstart(nj, bn)

            gwait(j, t)
            wstart(j, t)
        return carry

    lax.fori_loop(0, STEPS // NBUF, body, 0)

    for t in range(NBUF):
        wwait(STEPS - NBUF + t, t)


def kernel(indices, embed_weight):
    idx_t = indices.astype(jnp.int32).T
    out = _sc_gather(idx_t, embed_weight)
    return out.transpose(1, 0, 2)


# speedup vs baseline: 15.7992x; 2.3992x over previous
import functools

import jax
import jax.numpy as jnp
from jax import lax
from jax.experimental import pallas as pl
from jax.experimental.pallas import tpu as pltpu
from jax.experimental.pallas import tpu_sc as plsc

EMBED_DIM = 128
HIST = 50
BATCH = 4096
VOCAB = 1002
NUM_WORKERS = 32
BPW = BATCH // NUM_WORKERS
STEPS = HIST
NBUF = 5
LOOK = 3

_mesh = plsc.VectorSubcoreMesh(core_axis_name="c", subcore_axis_name="s")


@functools.partial(
    pl.kernel,
    mesh=_mesh,
    out_type=jax.ShapeDtypeStruct((HIST, BATCH, EMBED_DIM), jnp.float32),
    scratch_types=[
        pltpu.VMEM((HIST, BPW), jnp.int32),
        pltpu.VMEM((NBUF, BPW, EMBED_DIM), jnp.float32),
        pltpu.VMEM_SHARED((VOCAB, EMBED_DIM), jnp.float32),
        pltpu.SemaphoreType.DMA((NBUF,)),
        pltpu.SemaphoreType.DMA((NBUF,)),
    ],
)
def _sc_gather(idx_hbm, table_hbm, out_hbm, idx_v, bufs, table_sh, gsem, wsem):
    c = lax.axis_index("c")
    s = lax.axis_index("s")
    wid = s * 2 + c
    b0 = wid * BPW

    @pl.when(s == 0)
    def _():
        pltpu.sync_copy(table_hbm, table_sh)

    pltpu.sync_copy(idx_hbm.at[:, pl.ds(b0, BPW)], idx_v)
    plsc.subcore_barrier()

    def gstart(j, b):
        pltpu.async_copy(table_sh.at[idx_v.at[j]], bufs.at[b], gsem.at[b])

    def gwait(j, b):
        pltpu.make_async_copy(
            table_sh.at[idx_v.at[j]], bufs.at[b], gsem.at[b]
        ).wait()

    def wstart(j, b):
        pltpu.async_copy(
            bufs.at[b], out_hbm.at[j, pl.ds(b0, BPW)], wsem.at[b]
        )

    def wwait(j, b):
        pltpu.make_async_copy(
            bufs.at[b], out_hbm.at[j, pl.ds(b0, BPW)], wsem.at[b]
        ).wait()

    for t in range(LOOK):
        gstart(t, t)

    def body(jj, carry):
        j0 = jj * NBUF
        for t in range(NBUF):
            j = j0 + t
            bn = (t + LOOK) % NBUF
            nj = j + LOOK

            @pl.when(jnp.logical_and(nj < STEPS, j >= NBUF - LOOK))
            def _():
                wwait(nj - NBUF, bn)

            @pl.when(nj < STEPS)
            def _():
                gstart(nj, bn)

            gwait(j, t)
            wstart(j, t)
        return carry

    lax.fori_loop(0, STEPS // NBUF, body, 0)

    for t in range(NBUF):
        wwait(STEPS - NBUF + t, t)


def kernel(indices, embed_weight):
    idx_t = indices.astype(jnp.int32).T
    out = _sc_gather(idx_t, embed_weight)
    return out.transpose(1, 0, 2)
